# baseline (speedup 1.0000x reference)
import jax
import jax.numpy as jnp
from jax import lax
from jax.experimental import pallas as pl
from jax.experimental.pallas import tpu as pltpu
from jax.experimental.pallas import tpu_sc as plsc

SHIFT = 500000
B = 16384
D = 64
V = 1000000
L = 16
NC = 2
NS = 16
NW = NC * NS
BPW = B // NW
G = 64
NCH = BPW // G
TR = 8


def _rsqrt(x):
    i = lax.bitcast_convert_type(x, jnp.int32)
    i = jnp.int32(0x5F3759DF) - lax.shift_right_logical(i, 1)
    y = lax.bitcast_convert_type(i, jnp.float32)
    y = y * (1.5 - 0.5 * x * y * y)
    y = y * (1.5 - 0.5 * x * y * y)
    return y


def _body(offset_hbm, table_hbm, out_hbm, off_v, tidx_v, rmod_v,
          stage, sem0, sem1):
    wid = lax.axis_index("s") * NC + lax.axis_index("c")
    base = wid * BPW

    pltpu.sync_copy(offset_hbm.at[pl.ds(base, BPW)], off_v)
    for i in range(BPW // L):
        v = off_v[pl.ds(i * L, L)] + SHIFT
        tidx_v[pl.ds(i * L, L)] = lax.shift_right_logical(v, 3)
        rmod_v[pl.ds(i * L, L)] = v & 7

    def fire(c, sem):
        for g in range(G // L):
            tv = tidx_v[pl.ds(c * G + g * L, L)]
            rv = rmod_v[pl.ds(c * G + g * L, L)]
            for j in range(L):
                pltpu.async_copy(table_hbm.at[tv[j], rv[j]],
                                 stage.at[c * G + g * L + j], sem)

    def drain(c, sem):
        pltpu.make_async_copy(out_hbm.at[pl.ds(0, G)],
                              stage.at[pl.ds(c * G, G)], sem).wait()

    def process(c):
        for g in range(G // L):
            for j in range(L):
                row = c * G + g * L + j
                ck = [stage[row, pl.ds(k * L, L)] for k in range(D // L)]
                acc = ck[0] * ck[0]
                for k in range(1, D // L):
                    acc = acc + ck[k] * ck[k]
                s = jnp.sum(acc)
                sv = jnp.broadcast_to(s, (L,))
                scale = jnp.minimum(1.0, 2.0 * _rsqrt(sv))
                for k in range(D // L):
                    stage[row, pl.ds(k * L, L)] = ck[k] * scale

    fire(0, sem0)
    fire(1, sem1)

    @pl.loop(0, NCH, step=2)
    def _pair(c):
        drain(c, sem0)
        process(c)

        @pl.when(c + 2 < NCH)
        def _():
            fire(c + 2, sem0)

        drain(c + 1, sem1)
        process(c + 1)

        @pl.when(c + 3 < NCH)
        def _():
            fire(c + 3, sem1)

    pltpu.sync_copy(stage, out_hbm.at[pl.ds(base, BPW)])


@jax.jit
def kernel(offset, table):
    blocks = table.reshape(V // TR, TR, D)
    mesh = plsc.VectorSubcoreMesh(core_axis_name="c", subcore_axis_name="s",
                                  num_cores=NC, num_subcores=NS)
    run = pl.kernel(
        _body,
        out_type=jax.ShapeDtypeStruct((B, D), jnp.float32),
        mesh=mesh,
        scratch_types=[
            pltpu.VMEM((BPW,), jnp.int32),
            pltpu.VMEM((BPW,), jnp.int32),
            pltpu.VMEM((BPW,), jnp.int32),
            pltpu.VMEM((BPW, D), jnp.float32),
            pltpu.SemaphoreType.DMA,
            pltpu.SemaphoreType.DMA,
        ],
        compiler_params=pltpu.CompilerParams(needs_layout_passes=False,
                                             use_tc_tiling_on_sc=True),
    )
    return run(offset, blocks)


# speedup vs baseline: 1.0278x; 1.0278x over previous
import jax
import jax.numpy as jnp
from jax import lax
from jax.experimental import pallas as pl
from jax.experimental.pallas import tpu as pltpu
from jax.experimental.pallas import tpu_sc as plsc

SHIFT = 500000
B = 16384
D = 64
V = 1000000
L = 16
NC = 2
NS = 16
NW = NC * NS
BPW = B // NW
G = 32
NCH = BPW // G
TR = 8


def _rsqrt(x):
    i = lax.bitcast_convert_type(x, jnp.int32)
    i = jnp.int32(0x5F3759DF) - lax.shift_right_logical(i, 1)
    y = lax.bitcast_convert_type(i, jnp.float32)
    y = y * (1.5 - 0.5 * x * y * y)
    y = y * (1.5 - 0.5 * x * y * y)
    return y


def _body(offset_hbm, table_hbm, out_hbm, off_v, tidx_v, rmod_v,
          stage, sem0, sem1):
    wid = lax.axis_index("s") * NC + lax.axis_index("c")
    base = wid * BPW

    pltpu.sync_copy(offset_hbm.at[pl.ds(base, BPW)], off_v)
    for i in range(BPW // L):
        v = off_v[pl.ds(i * L, L)] + SHIFT
        tidx_v[pl.ds(i * L, L)] = lax.shift_right_logical(v, 3)
        rmod_v[pl.ds(i * L, L)] = v & 7

    def fire(c, sem):
        for g in range(G // L):
            tv = tidx_v[pl.ds(c * G + g * L, L)]
            rv = rmod_v[pl.ds(c * G + g * L, L)]
            for j in range(L):
                pltpu.async_copy(table_hbm.at[tv[j], rv[j]],
                                 stage.at[c * G + g * L + j], sem)

    def drain(c, sem):
        pltpu.make_async_copy(out_hbm.at[pl.ds(0, G)],
                              stage.at[pl.ds(c * G, G)], sem).wait()

    def process(c):
        for g in range(G // L):
            for j in range(L):
                row = c * G + g * L + j
                ck = [stage[row, pl.ds(k * L, L)] for k in range(D // L)]
                acc = ck[0] * ck[0]
                for k in range(1, D // L):
                    acc = acc + ck[k] * ck[k]
                s = jnp.sum(acc)
                sv = jnp.broadcast_to(s, (L,))
                scale = jnp.minimum(1.0, 2.0 * _rsqrt(sv))
                for k in range(D // L):
                    stage[row, pl.ds(k * L, L)] = ck[k] * scale

    fire(0, sem0)
    fire(1, sem1)

    @pl.loop(0, NCH, step=2)
    def _pair(c):
        drain(c, sem0)
        process(c)

        @pl.when(c + 2 < NCH)
        def _():
            fire(c + 2, sem0)

        drain(c + 1, sem1)
        process(c + 1)

        @pl.when(c + 3 < NCH)
        def _():
            fire(c + 3, sem1)

    pltpu.sync_copy(stage, out_hbm.at[pl.ds(base, BPW)])


@jax.jit
def kernel(offset, table):
    blocks = table.reshape(V // TR, TR, D)
    mesh = plsc.VectorSubcoreMesh(core_axis_name="c", subcore_axis_name="s",
                                  num_cores=NC, num_subcores=NS)
    run = pl.kernel(
        _body,
        out_type=jax.ShapeDtypeStruct((B, D), jnp.float32),
        mesh=mesh,
        scratch_types=[
            pltpu.VMEM((BPW,), jnp.int32),
            pltpu.VMEM((BPW,), jnp.int32),
            pltpu.VMEM((BPW,), jnp.int32),
            pltpu.VMEM((BPW, D), jnp.float32),
            pltpu.SemaphoreType.DMA,
            pltpu.SemaphoreType.DMA,
        ],
        compiler_params=pltpu.CompilerParams(needs_layout_passes=False,
                                             use_tc_tiling_on_sc=True),
    )
    return run(offset, blocks)
